# baseline (speedup 1.0000x reference)
import jax
import jax.numpy as jnp
from jax import lax
from jax.experimental import pallas as pl
from jax.experimental.pallas import tpu as pltpu
from jax.experimental.pallas import tpu_sc as plsc

N = 10000
E = 160000
D = 256
EDIM = 7
H = 128
NSUB = 16
EPS = E // NSUB
B = 80
NCH = 124
TAIL = EPS - NCH * B
RPS = 624
RTAIL = N - NSUB * RPS


def _tc_node_body(nfeat_ref, wl_ref, bl_ref, be_ref, root_ref, degs_ref,
                  hb_ref, rst0_ref):
    h = jnp.dot(nfeat_ref[...], wl_ref[...], preferred_element_type=jnp.float32)
    h = h + bl_ref[...]
    hb_ref[...] = h + be_ref[...]
    rst0_ref[...] = jnp.maximum(h + root_ref[...], 0.0) / degs_ref[...]


def _tc_node(nfeat, Wl, bl2, be2, root_emb, degs):
    R = 2000
    grid = (2, N // R)
    return pl.pallas_call(
        _tc_node_body,
        grid=grid,
        in_specs=[
            pl.BlockSpec((R, D), lambda j, i: (i, 0)),
            pl.BlockSpec((D, H), lambda j, i: (0, j)),
            pl.BlockSpec((1, H), lambda j, i: (0, j)),
            pl.BlockSpec((1, H), lambda j, i: (0, j)),
            pl.BlockSpec((1, H), lambda j, i: (0, j)),
            pl.BlockSpec((R, 1), lambda j, i: (i, 0)),
        ],
        out_specs=[
            pl.BlockSpec((R, H), lambda j, i: (j * (N // R) + i, 0)),
            pl.BlockSpec((R, H), lambda j, i: (j * (N // R) + i, 0)),
        ],
        out_shape=[
            jax.ShapeDtypeStruct((2 * N, H), jnp.float32),
            jax.ShapeDtypeStruct((2 * N, H), jnp.float32),
        ],
    )(nfeat, Wl, bl2, be2, root_emb, degs)


def _sc_body(hb, weh, rst0, efp, srci, dsti, out,
             acc,
             srcv0, srcv1, dstv0, dstv1,
             hrows0, hrows1, efv0, efv1,
             sbuf0, sbuf1, wvm,
             sl0, sl1, sg0, sg1, st):
    c = lax.axis_index("c")
    s = lax.axis_index("s")
    cN = c * N
    cE = c * E
    r0 = s * RPS
    e0 = s * EPS

    srcv = (srcv0, srcv1)
    dstv = (dstv0, dstv1)
    hrows = (hrows0, hrows1)
    efv = (efv0, efv1)
    sbuf = (sbuf0, sbuf1)
    sl = (sl0, sl1)
    sg = (sg0, sg1)

    pltpu.sync_copy(weh.at[pl.ds(c * 8, 8)], wvm)

    pltpu.sync_copy(rst0.at[pl.ds(cN + r0, RPS)], acc.at[pl.ds(r0, RPS)])

    @pl.when(s == 0)
    def _():
        pltpu.sync_copy(rst0.at[pl.ds(cN + NSUB * RPS, RTAIL)],
                        acc.at[pl.ds(NSUB * RPS, RTAIL)])

    plsc.subcore_barrier()

    def issue_loads(t, b):
        base = e0 + t * B
        pltpu.async_copy(srci.at[pl.ds(cE + base, B)], srcv[b], sl[b])
        pltpu.async_copy(dsti.at[pl.ds(base, B)], dstv[b], sl[b])
        pltpu.async_copy(efp.at[pl.ds(base * 8, B * 8)], efv[b], sl[b])

    def wait_loads(t, b):
        base = e0 + t * B
        pltpu.make_async_copy(srci.at[pl.ds(cE + base, B)], srcv[b], sl[b]).wait()
        pltpu.make_async_copy(dsti.at[pl.ds(base, B)], dstv[b], sl[b]).wait()
        pltpu.make_async_copy(efp.at[pl.ds(base * 8, B * 8)], efv[b], sl[b]).wait()

    def compute_chunk(hr, ef, sb):
        for p in range(4):
            wv = [[wvm[k, pl.ds((p * 2 + q) * 16, 16)] for q in range(2)]
                  for k in range(EDIM)]

            def edge_pair(j2, c2):
                fv = ef[pl.ds(j2 * 16, 16)]
                for half in range(2):
                    e = 2 * j2 + half
                    lane = 8 * half
                    nsc = fv[lane + 7]
                    sk = [nsc * fv[lane + k] for k in range(EDIM)]
                    for q in range(2):
                        col = (p * 2 + q) * 16
                        acc_v = nsc * hr[e, pl.ds(col, 16)]
                        for k in range(EDIM):
                            acc_v = acc_v + sk[k] * wv[k][q]
                        sb[e, pl.ds(col, 16)] = jnp.maximum(acc_v, 0.0)
                return c2

            lax.fori_loop(0, B // 2, edge_pair, 0)

    issue_loads(0, 0)
    issue_loads(1, 1)
    wait_loads(0, 0)
    pltpu.async_copy(hb.at[srcv[0]], hrows[0], sg[0])

    def outer(o, carry):
        for b in (0, 1):
            t = 2 * o + b
            nb = 1 - b

            @pl.when(t + 1 < NCH)
            def _():
                wait_loads(t + 1, nb)
                pltpu.async_copy(hb.at[srcv[nb]], hrows[nb], sg[nb])

            pltpu.make_async_copy(hb.at[srcv[b]], hrows[b], sg[b]).wait()
            compute_chunk(hrows[b], efv[b], sbuf[b])
            pltpu.sync_copy(sbuf[b], acc.at[dstv[b]], add=True)

            @pl.when(t + 2 < NCH)
            def _():
                issue_loads(t + 2, b)
        return carry

    lax.fori_loop(0, NCH // 2, outer, 0)

    tb = e0 + NCH * B
    pltpu.sync_copy(srci.at[pl.ds(cE + tb, B)], srcv0)
    pltpu.sync_copy(dsti.at[pl.ds(tb, B)], dstv0)
    pltpu.sync_copy(efp.at[pl.ds(tb * 8, B * 8)], efv0)
    pltpu.async_copy(hb.at[srcv0], hrows0, st).wait()
    compute_chunk(hrows0, efv0, sbuf0)
    pltpu.sync_copy(sbuf0, acc.at[dstv0], add=True)

    plsc.subcore_barrier()
    pltpu.sync_copy(acc.at[pl.ds(r0, RPS)], out.at[pl.ds(cN + r0, RPS)])

    @pl.when(s == 0)
    def _():
        pltpu.sync_copy(acc.at[pl.ds(NSUB * RPS, RTAIL)],
                        out.at[pl.ds(cN + NSUB * RPS, RTAIL)])


def _sc_edge_aggregate(hb, weh, rst0, efp, srcN2, dst):
    mesh = plsc.VectorSubcoreMesh(core_axis_name="c", subcore_axis_name="s")
    return pl.kernel(
        _sc_body,
        out_type=jax.ShapeDtypeStruct((2 * N, H), jnp.float32),
        mesh=mesh,
        scratch_types=[
            pltpu.VMEM_SHARED((N, H), jnp.float32),
            pltpu.VMEM((B,), jnp.int32), pltpu.VMEM((B,), jnp.int32),
            pltpu.VMEM((B,), jnp.int32), pltpu.VMEM((B,), jnp.int32),
            pltpu.VMEM((B, H), jnp.float32), pltpu.VMEM((B, H), jnp.float32),
            pltpu.VMEM((B * 8,), jnp.float32), pltpu.VMEM((B * 8,), jnp.float32),
            pltpu.VMEM((B, H), jnp.float32), pltpu.VMEM((B, H), jnp.float32),
            pltpu.VMEM((8, H), jnp.float32),
            pltpu.SemaphoreType.DMA, pltpu.SemaphoreType.DMA,
            pltpu.SemaphoreType.DMA, pltpu.SemaphoreType.DMA,
            pltpu.SemaphoreType.DMA,
        ],
    )(hb, weh, rst0, efp, srcN2, dst)


def kernel(nfeat, efeat, degs, norm, edge_index, Wl, bl, We, be, root_emb):
    hb, rst0 = _tc_node(nfeat, Wl, bl.reshape(1, D), be.reshape(1, D),
                        root_emb, degs)
    zrow = jnp.zeros((1, H), jnp.float32)
    weh = jnp.concatenate([We[:, :H], zrow, We[:, H:], zrow])
    efp = jnp.concatenate([efeat, norm], axis=1).reshape(E * 8)
    src = edge_index[0]
    srcN2 = jnp.concatenate([src, src + N])
    out = _sc_edge_aggregate(hb, weh, rst0, efp, srcN2, edge_index[1])
    out2 = out.reshape(2, N, H)
    return jnp.concatenate([out2[0], out2[1]], axis=1)


# speedup vs baseline: 1.2315x; 1.2315x over previous
import jax
import jax.numpy as jnp
from jax import lax
from jax.experimental import pallas as pl
from jax.experimental.pallas import tpu as pltpu
from jax.experimental.pallas import tpu_sc as plsc

N = 10000
E = 160000
D = 256
EDIM = 7
H = 128
NSUB = 16
EPS = E // NSUB
B = 80
NCH = 124
TAIL = EPS - NCH * B
RPS = 624
RTAIL = N - NSUB * RPS


def _tc_node_body(nfeat_ref, wl_ref, bl_ref, be_ref, root_ref, degs_ref,
                  hb_ref, rst0_ref):
    h = jnp.dot(nfeat_ref[...], wl_ref[...], preferred_element_type=jnp.float32)
    h = h + bl_ref[...]
    hb_ref[...] = h + be_ref[...]
    rst0_ref[...] = jnp.maximum(h + root_ref[...], 0.0) / degs_ref[...]


def _tc_node(nfeat, Wl, bl2, be2, root_emb, degs):
    R = 2000
    grid = (2, N // R)
    return pl.pallas_call(
        _tc_node_body,
        grid=grid,
        in_specs=[
            pl.BlockSpec((R, D), lambda j, i: (i, 0)),
            pl.BlockSpec((D, H), lambda j, i: (0, j)),
            pl.BlockSpec((1, H), lambda j, i: (0, j)),
            pl.BlockSpec((1, H), lambda j, i: (0, j)),
            pl.BlockSpec((1, H), lambda j, i: (0, j)),
            pl.BlockSpec((R, 1), lambda j, i: (i, 0)),
        ],
        out_specs=[
            pl.BlockSpec((R, H), lambda j, i: (j * (N // R) + i, 0)),
            pl.BlockSpec((R, H), lambda j, i: (j * (N // R) + i, 0)),
        ],
        out_shape=[
            jax.ShapeDtypeStruct((2 * N, H), jnp.float32),
            jax.ShapeDtypeStruct((2 * N, H), jnp.float32),
        ],
    )(nfeat, Wl, bl2, be2, root_emb, degs)


def _sc_body(hb, weh, rst0, efp, srci, dsti, out,
             acc,
             srcv0, srcv1, dstv0, dstv1,
             hrows0, hrows1, efv0, efv1,
             sbuf0, sbuf1, wvm,
             sl0, sl1, sg0, sg1, st):
    c = lax.axis_index("c")
    s = lax.axis_index("s")
    cN = c * N
    cE = c * E
    r0 = s * RPS
    e0 = s * EPS

    srcv = (srcv0, srcv1)
    dstv = (dstv0, dstv1)
    hrows = (hrows0, hrows1)
    efv = (efv0, efv1)
    sbuf = (sbuf0, sbuf1)
    sl = (sl0, sl1)
    sg = (sg0, sg1)

    pltpu.sync_copy(weh.at[pl.ds(c * 8, 8)], wvm)

    pltpu.sync_copy(rst0.at[pl.ds(cN + r0, RPS)], acc.at[pl.ds(r0, RPS)])

    @pl.when(s == 0)
    def _():
        pltpu.sync_copy(rst0.at[pl.ds(cN + NSUB * RPS, RTAIL)],
                        acc.at[pl.ds(NSUB * RPS, RTAIL)])

    plsc.subcore_barrier()

    def issue_loads(t, b):
        base = e0 + t * B
        pltpu.async_copy(srci.at[pl.ds(cE + base, B)], srcv[b], sl[b])
        pltpu.async_copy(dsti.at[pl.ds(base, B)], dstv[b], sl[b])
        pltpu.async_copy(efp.at[pl.ds(base * 8, B * 8)], efv[b], sl[b])

    def wait_loads(t, b):
        base = e0 + t * B
        pltpu.make_async_copy(srci.at[pl.ds(cE + base, B)], srcv[b], sl[b]).wait()
        pltpu.make_async_copy(dsti.at[pl.ds(base, B)], dstv[b], sl[b]).wait()
        pltpu.make_async_copy(efp.at[pl.ds(base * 8, B * 8)], efv[b], sl[b]).wait()

    def compute_chunk(hr, ef, sb):
        for p in range(2):
            wv = [[wvm[k, pl.ds((p * 4 + q) * 16, 16)] for q in range(4)]
                  for k in range(EDIM)]

            def edge_pair(j2, c2):
                fv = ef[pl.ds(j2 * 16, 16)]
                for half in range(2):
                    e = 2 * j2 + half
                    lane = 8 * half
                    nsc = fv[lane + 7]
                    sk = [nsc * fv[lane + k] for k in range(EDIM)]
                    for q in range(4):
                        col = (p * 4 + q) * 16
                        acc_v = nsc * hr[e, pl.ds(col, 16)]
                        for k in range(EDIM):
                            acc_v = acc_v + sk[k] * wv[k][q]
                        sb[e, pl.ds(col, 16)] = jnp.maximum(acc_v, 0.0)
                return c2

            lax.fori_loop(0, B // 2, edge_pair, 0)

    issue_loads(0, 0)
    issue_loads(1, 1)
    wait_loads(0, 0)
    pltpu.async_copy(hb.at[srcv[0]], hrows[0], sg[0])

    def outer(o, carry):
        for b in (0, 1):
            t = 2 * o + b
            nb = 1 - b

            @pl.when(t + 1 < NCH)
            def _():
                wait_loads(t + 1, nb)
                pltpu.async_copy(hb.at[srcv[nb]], hrows[nb], sg[nb])

            pltpu.make_async_copy(hb.at[srcv[b]], hrows[b], sg[b]).wait()
            compute_chunk(hrows[b], efv[b], sbuf[b])
            pltpu.sync_copy(sbuf[b], acc.at[dstv[b]], add=True)

            @pl.when(t + 2 < NCH)
            def _():
                issue_loads(t + 2, b)
        return carry

    lax.fori_loop(0, NCH // 2, outer, 0)

    tb = e0 + NCH * B
    pltpu.sync_copy(srci.at[pl.ds(cE + tb, B)], srcv0)
    pltpu.sync_copy(dsti.at[pl.ds(tb, B)], dstv0)
    pltpu.sync_copy(efp.at[pl.ds(tb * 8, B * 8)], efv0)
    pltpu.async_copy(hb.at[srcv0], hrows0, st).wait()
    compute_chunk(hrows0, efv0, sbuf0)
    pltpu.sync_copy(sbuf0, acc.at[dstv0], add=True)

    plsc.subcore_barrier()
    pltpu.sync_copy(acc.at[pl.ds(r0, RPS)], out.at[pl.ds(cN + r0, RPS)])

    @pl.when(s == 0)
    def _():
        pltpu.sync_copy(acc.at[pl.ds(NSUB * RPS, RTAIL)],
                        out.at[pl.ds(cN + NSUB * RPS, RTAIL)])


def _sc_edge_aggregate(hb, weh, rst0, efp, srcN2, dst):
    mesh = plsc.VectorSubcoreMesh(core_axis_name="c", subcore_axis_name="s")
    return pl.kernel(
        _sc_body,
        out_type=jax.ShapeDtypeStruct((2 * N, H), jnp.float32),
        mesh=mesh,
        scratch_types=[
            pltpu.VMEM_SHARED((N, H), jnp.float32),
            pltpu.VMEM((B,), jnp.int32), pltpu.VMEM((B,), jnp.int32),
            pltpu.VMEM((B,), jnp.int32), pltpu.VMEM((B,), jnp.int32),
            pltpu.VMEM((B, H), jnp.float32), pltpu.VMEM((B, H), jnp.float32),
            pltpu.VMEM((B * 8,), jnp.float32), pltpu.VMEM((B * 8,), jnp.float32),
            pltpu.VMEM((B, H), jnp.float32), pltpu.VMEM((B, H), jnp.float32),
            pltpu.VMEM((8, H), jnp.float32),
            pltpu.SemaphoreType.DMA, pltpu.SemaphoreType.DMA,
            pltpu.SemaphoreType.DMA, pltpu.SemaphoreType.DMA,
            pltpu.SemaphoreType.DMA,
        ],
    )(hb, weh, rst0, efp, srcN2, dst)


def kernel(nfeat, efeat, degs, norm, edge_index, Wl, bl, We, be, root_emb):
    hb, rst0 = _tc_node(nfeat, Wl, bl.reshape(1, D), be.reshape(1, D),
                        root_emb, degs)
    zrow = jnp.zeros((1, H), jnp.float32)
    weh = jnp.concatenate([We[:, :H], zrow, We[:, H:], zrow])
    efp = jnp.concatenate([efeat, norm], axis=1).reshape(E * 8)
    src = edge_index[0]
    srcN2 = jnp.concatenate([src, src + N])
    out = _sc_edge_aggregate(hb, weh, rst0, efp, srcN2, edge_index[1])
    out2 = out.reshape(2, N, H)
    return jnp.concatenate([out2[0], out2[1]], axis=1)
